# baseline (speedup 1.0000x reference)
import jax
import jax.numpy as jnp
from jax.experimental import pallas as pl
from jax.experimental.pallas import tpu as pltpu
from jax.experimental.pallas import tpu_sc as plsc

_W = 64
_IDX_BLK = 1024
_NTILES = 32


def kernel(indices, weight):
    B, L = indices.shape
    V, D = weight.shape
    N = B * L
    flat_idx = indices.reshape(N)

    rows_per_tile = N // _NTILES
    blocks_per_tile = rows_per_tile // _IDX_BLK
    win_per_blk = _IDX_BLK // _W

    mesh = plsc.VectorSubcoreMesh(core_axis_name="core",
                                  subcore_axis_name="subcore")

    @pl.kernel(
        out_type=jax.ShapeDtypeStruct((N, D), weight.dtype),
        mesh=mesh,
        scratch_types=[
            pltpu.VMEM((2, _IDX_BLK), jnp.int32),
            pltpu.VMEM((2, _W, D), weight.dtype),
            pltpu.SemaphoreType.DMA,
            pltpu.SemaphoreType.DMA,
            pltpu.SemaphoreType.DMA,
            pltpu.SemaphoreType.DMA,
            pltpu.SemaphoreType.DMA,
            pltpu.SemaphoreType.DMA,
        ],
    )
    def sc_gather(i_hbm, w_hbm, o_hbm, idxb, rows,
                  isem0, isem1, gsem0, gsem1, wsem0, wsem1):
        isems = [isem0, isem1]
        gsems = [gsem0, gsem1]
        wsems = [wsem0, wsem1]

        wid = (jax.lax.axis_index("subcore") * 2
               + jax.lax.axis_index("core"))
        base = wid * rows_per_tile

        pltpu.async_copy(i_hbm.at[pl.ds(base, _IDX_BLK)], idxb.at[0],
                         isems[0])

        @pl.loop(0, blocks_per_tile, step=2)
        def _(g):
            for p in range(2):
                blk = g + p
                blk_base = base + blk * _IDX_BLK

                @pl.when(blk + 1 < blocks_per_tile)
                def _():
                    pltpu.async_copy(
                        i_hbm.at[pl.ds(blk_base + _IDX_BLK, _IDX_BLK)],
                        idxb.at[1 - p], isems[1 - p])

                pltpu.make_async_copy(
                    i_hbm.at[pl.ds(blk_base, _IDX_BLK)], idxb.at[p],
                    isems[p]).wait()

                @pl.loop(0, win_per_blk, step=2)
                def _(kk):
                    for b in range(2):
                        k = kk + b
                        row0 = blk_base + k * _W

                        @pl.when(blk * win_per_blk + k >= 2)
                        def _():
                            pltpu.make_async_copy(
                                rows.at[b], o_hbm.at[pl.ds(row0, _W)],
                                wsems[b]).wait()

                        pltpu.async_copy(
                            w_hbm.at[idxb.at[p, pl.ds(k * _W, _W)]],
                            rows.at[b], gsems[b]).wait()

                        pltpu.async_copy(rows.at[b],
                                         o_hbm.at[pl.ds(row0, _W)],
                                         wsems[b])

        for b in range(2):
            last_row0 = base
            pltpu.make_async_copy(rows.at[b],
                                  o_hbm.at[pl.ds(last_row0, _W)],
                                  wsems[b]).wait()

    out = sc_gather(flat_idx, weight)
    return out.reshape(B, L, D)
